# baseline (speedup 1.0000x reference)
import functools

import jax
import jax.numpy as jnp
from jax import lax
from jax.experimental import pallas as pl
from jax.experimental.pallas import tpu as pltpu
from jax.experimental.pallas import tpu_sc as plsc

NC = 2
NS = 16
L = 16
NW = NC * NS
SC_CHUNK = 16384


def _sc_add(x_flat, e_flat, x_off, n_elems, e_size):
    per_w = n_elems // NW
    n_chunks = per_w // SC_CHUNK
    n_half = n_chunks // 2

    def body(x_hbm, e_hbm, o_hbm, ib, eb, ob, isem, esem, osem):
        wid = lax.axis_index("s") * NC + lax.axis_index("c")
        base = wid * per_w

        def start_in(g, b):
            off = base + g * SC_CHUNK
            pltpu.make_async_copy(
                x_hbm.at[pl.ds(x_off + off, SC_CHUNK)], ib.at[b],
                isem.at[b]).start()
            pltpu.make_async_copy(
                e_hbm.at[pl.ds(lax.rem(x_off + off, e_size), SC_CHUNK)],
                eb.at[b], esem.at[b]).start()

        def wait_in(b):
            pltpu.make_async_copy(
                x_hbm.at[pl.ds(0, SC_CHUNK)], ib.at[b], isem.at[b]).wait()
            pltpu.make_async_copy(
                e_hbm.at[pl.ds(0, SC_CHUNK)], eb.at[b], esem.at[b]).wait()

        def out_copy(g, b):
            return pltpu.make_async_copy(
                ob.at[b], o_hbm.at[pl.ds(base + g * SC_CHUNK, SC_CHUNK)],
                osem.at[b])

        start_in(0, 0)
        start_in(1, 1)

        def step(i, carry):
            for b in range(2):
                g = 2 * i + b
                wait_in(b)

                @pl.when(i >= 1)
                def _():
                    out_copy(0, b).wait()

                ibb, ebb, obb = ib.at[b], eb.at[b], ob.at[b]

                def compute(k, c):
                    for u in range(4):
                        sl = pl.ds((k * 4 + u) * L, L)
                        obb[sl] = ibb[sl] + ebb[sl]
                    return c

                lax.fori_loop(0, SC_CHUNK // L // 4, compute, 0)
                out_copy(g, b).start()

                @pl.when(i + 1 < n_half)
                def _():
                    start_in(g + 2, b)
            return carry

        lax.fori_loop(0, n_half, step, 0)
        out_copy(0, 0).wait()
        out_copy(0, 1).wait()

    mesh = plsc.VectorSubcoreMesh(core_axis_name="c", subcore_axis_name="s")
    f = pl.kernel(
        body,
        out_type=jax.ShapeDtypeStruct((n_elems,), jnp.float32),
        mesh=mesh,
        scratch_types=[
            pltpu.VMEM((2, SC_CHUNK), jnp.float32),
            pltpu.VMEM((2, SC_CHUNK), jnp.float32),
            pltpu.VMEM((2, SC_CHUNK), jnp.float32),
            pltpu.SemaphoreType.DMA((2,)),
            pltpu.SemaphoreType.DMA((2,)),
            pltpu.SemaphoreType.DMA((2,)),
        ],
    )
    return f(x_flat, e_flat)


def kernel(inputs, embeddings):
    b, s, d = inputs.shape
    e_size = s * d
    n_elems = b * e_size
    x_flat = inputs.reshape(n_elems)
    e_flat = embeddings[:s].reshape(e_size)
    out_flat = _sc_add(x_flat, e_flat, 0, n_elems, e_size)
    return out_flat.reshape(b, s, d)


# speedup vs baseline: 1.3167x; 1.3167x over previous
import functools

import jax
import jax.numpy as jnp
from jax import lax
from jax.experimental import pallas as pl
from jax.experimental.pallas import tpu as pltpu
from jax.experimental.pallas import tpu_sc as plsc

NC = 2
NS = 16
L = 16
NW = NC * NS
SC_CHUNK = 16384


def _sc_add(x_flat, e_flat, x_off, n_elems, e_size):
    per_w = n_elems // NW
    n_chunks = per_w // SC_CHUNK
    n_half = n_chunks // 2

    def body(x_hbm, e_hbm, o_hbm, ib, eb, ob, isem, esem, osem):
        wid = lax.axis_index("s") * NC + lax.axis_index("c")
        base = wid * per_w

        def start_in(g, b):
            off = base + g * SC_CHUNK
            pltpu.make_async_copy(
                x_hbm.at[pl.ds(x_off + off, SC_CHUNK)], ib.at[b],
                isem.at[b]).start()
            pltpu.make_async_copy(
                e_hbm.at[pl.ds(lax.rem(x_off + off, e_size), SC_CHUNK)],
                eb.at[b], esem.at[b]).start()

        def wait_in(b):
            pltpu.make_async_copy(
                x_hbm.at[pl.ds(0, SC_CHUNK)], ib.at[b], isem.at[b]).wait()
            pltpu.make_async_copy(
                e_hbm.at[pl.ds(0, SC_CHUNK)], eb.at[b], esem.at[b]).wait()

        def out_copy(g, b):
            return pltpu.make_async_copy(
                ob.at[b], o_hbm.at[pl.ds(base + g * SC_CHUNK, SC_CHUNK)],
                osem.at[b])

        start_in(0, 0)
        start_in(1, 1)

        def step(i, carry):
            for b in range(2):
                g = 2 * i + b
                wait_in(b)

                @pl.when(i >= 1)
                def _():
                    out_copy(0, b).wait()

                ibb, ebb, obb = ib.at[b], eb.at[b], ob.at[b]

                def compute(k, c):
                    for u in range(4):
                        sl = pl.ds((k * 4 + u) * L, L)
                        obb[sl] = ibb[sl] + ebb[sl]
                    return c

                lax.fori_loop(0, SC_CHUNK // L // 4, compute, 0)
                out_copy(g, b).start()

                @pl.when(i + 1 < n_half)
                def _():
                    start_in(g + 2, b)
            return carry

        lax.fori_loop(0, n_half, step, 0)
        out_copy(0, 0).wait()
        out_copy(0, 1).wait()

    mesh = plsc.VectorSubcoreMesh(core_axis_name="c", subcore_axis_name="s")
    f = pl.kernel(
        body,
        out_type=jax.ShapeDtypeStruct((n_elems,), jnp.float32),
        mesh=mesh,
        scratch_types=[
            pltpu.VMEM((2, SC_CHUNK), jnp.float32),
            pltpu.VMEM((2, SC_CHUNK), jnp.float32),
            pltpu.VMEM((2, SC_CHUNK), jnp.float32),
            pltpu.SemaphoreType.DMA((2,)),
            pltpu.SemaphoreType.DMA((2,)),
            pltpu.SemaphoreType.DMA((2,)),
        ],
    )
    return f(x_flat, e_flat)


CS = 512
NBUF = 4
TC_B = 3


def _tc_body(x_hbm, e_hbm, o_hbm, in_buf, out_buf, emb_buf, in_sem, out_sem,
             emb_sem):
    _, s, d = x_hbm.shape
    n_s = s // CS
    total = TC_B * n_s

    def in_copy(t, slot):
        bi = t // n_s
        si = lax.rem(t, n_s)
        return pltpu.make_async_copy(
            x_hbm.at[bi, pl.ds(si * CS, CS), :], in_buf.at[slot],
            in_sem.at[slot])

    def out_copy(t, slot):
        bi = t // n_s
        si = lax.rem(t, n_s)
        return pltpu.make_async_copy(
            out_buf.at[slot], o_hbm.at[bi, pl.ds(si * CS, CS), :],
            out_sem.at[slot])

    for c in range(n_s):
        pltpu.make_async_copy(
            e_hbm.at[pl.ds(c * CS, CS), :],
            emb_buf.at[pl.ds(c * CS, CS), :], emb_sem.at[c]).start()
    for k in range(NBUF):
        in_copy(k, k).start()

    def step(t, carry):
        slot = lax.rem(t, NBUF)
        si = lax.rem(t, n_s)
        in_copy(t, slot).wait()

        @pl.when(t < n_s)
        def _():
            pltpu.make_async_copy(
                e_hbm.at[pl.ds(0, CS), :], emb_buf.at[pl.ds(0, CS), :],
                emb_sem.at[si]).wait()

        @pl.when(t >= NBUF)
        def _():
            out_copy(t - NBUF, slot).wait()

        out_buf[slot] = in_buf[slot] + emb_buf[pl.ds(si * CS, CS), :]
        out_copy(t, slot).start()

        @pl.when(t + NBUF < total)
        def _():
            in_copy(t + NBUF, slot).start()

        return carry

    lax.fori_loop(0, total, step, 0)

    for k in range(NBUF):
        slot = (total - NBUF + k) % NBUF
        out_copy(total - NBUF + k, slot).wait()


def _tc_add(inputs, emb):
    _, s, d = inputs.shape
    return pl.pallas_call(
        _tc_body,
        in_specs=[
            pl.BlockSpec(memory_space=pl.ANY),
            pl.BlockSpec(memory_space=pl.ANY),
        ],
        out_specs=pl.BlockSpec(memory_space=pl.ANY),
        out_shape=jax.ShapeDtypeStruct((TC_B, s, d), inputs.dtype),
        scratch_shapes=[
            pltpu.VMEM((NBUF, CS, d), jnp.float32),
            pltpu.VMEM((NBUF, CS, d), jnp.float32),
            pltpu.VMEM((s, d), jnp.float32),
            pltpu.SemaphoreType.DMA((NBUF,)),
            pltpu.SemaphoreType.DMA((NBUF,)),
            pltpu.SemaphoreType.DMA((s // CS,)),
        ],
    )(inputs, emb)


def kernel(inputs, embeddings):
    b, s, d = inputs.shape
    e_size = s * d
    n_elems = b * e_size
    emb = embeddings[:s]
    x_flat = inputs.reshape(n_elems)
    e_flat = emb.reshape(e_size)
    sc_n = (b - TC_B) * e_size
    sc_out = _sc_add(x_flat, e_flat, TC_B * e_size, sc_n, e_size)
    tc_out = _tc_add(inputs, emb)
    return jnp.concatenate(
        [tc_out, sc_out.reshape(b - TC_B, s, d)], axis=0)


# speedup vs baseline: 2.3369x; 1.7748x over previous
import jax
import jax.numpy as jnp
from jax import lax
from jax.experimental import pallas as pl
from jax.experimental.pallas import tpu as pltpu
from jax.experimental.pallas import tpu_sc as plsc

NC = 2
NS = 16
L = 16
NW = NC * NS
R = 16

CS = 512
NBUF = 4
TC_B = 3


def _sc_add_rows(inputs, emb, bi):
    _, s, d = inputs.shape
    rows_per_w = s // NW
    n_chunks = rows_per_w // R
    n_half = n_chunks // 2

    def body(x_hbm, e_hbm, o_hbm, ib, eb, ob, isem, esem, osem):
        wid = lax.axis_index("s") * NC + lax.axis_index("c")
        base = wid * rows_per_w

        def start_in(g, b):
            r0 = base + g * R
            pltpu.make_async_copy(
                x_hbm.at[bi, pl.ds(r0, R), :], ib.at[b], isem.at[b]).start()
            pltpu.make_async_copy(
                e_hbm.at[pl.ds(r0, R), :], eb.at[b], esem.at[b]).start()

        def wait_in(b):
            pltpu.make_async_copy(
                x_hbm.at[bi, pl.ds(0, R), :], ib.at[b], isem.at[b]).wait()
            pltpu.make_async_copy(
                e_hbm.at[pl.ds(0, R), :], eb.at[b], esem.at[b]).wait()

        def out_copy(g, b):
            return pltpu.make_async_copy(
                ob.at[b], o_hbm.at[0, pl.ds(base + g * R, R), :], osem.at[b])

        start_in(0, 0)
        start_in(1, 1)

        def step(i, carry):
            for b in range(2):
                g = 2 * i + b
                wait_in(b)

                @pl.when(i >= 1)
                def _():
                    out_copy(0, b).wait()

                ibb, ebb, obb = ib.at[b], eb.at[b], ob.at[b]

                def compute(k, c):
                    sl = pl.ds(k * L, L)
                    for row in range(R):
                        obb[row, sl] = ibb[row, sl] + ebb[row, sl]
                    return c

                lax.fori_loop(0, d // L, compute, 0)
                out_copy(g, b).start()

                @pl.when(i + 1 < n_half)
                def _():
                    start_in(g + 2, b)
            return carry

        lax.fori_loop(0, n_half, step, 0)
        out_copy(0, 0).wait()
        out_copy(0, 1).wait()

    mesh = plsc.VectorSubcoreMesh(core_axis_name="c", subcore_axis_name="s")
    f = pl.kernel(
        body,
        out_type=jax.ShapeDtypeStruct((1, s, d), jnp.float32),
        mesh=mesh,
        scratch_types=[
            pltpu.VMEM((2, R, d), jnp.float32),
            pltpu.VMEM((2, R, d), jnp.float32),
            pltpu.VMEM((2, R, d), jnp.float32),
            pltpu.SemaphoreType.DMA((2,)),
            pltpu.SemaphoreType.DMA((2,)),
            pltpu.SemaphoreType.DMA((2,)),
        ],
    )
    return f(inputs, emb)


def _tc_body(x_hbm, e_hbm, o_hbm, in_buf, out_buf, emb_buf, in_sem, out_sem,
             emb_sem):
    _, s, d = x_hbm.shape
    n_s = s // CS
    total = TC_B * n_s

    def in_copy(t, slot):
        bi = t // n_s
        si = lax.rem(t, n_s)
        return pltpu.make_async_copy(
            x_hbm.at[bi, pl.ds(si * CS, CS), :], in_buf.at[slot],
            in_sem.at[slot])

    def out_copy(t, slot):
        bi = t // n_s
        si = lax.rem(t, n_s)
        return pltpu.make_async_copy(
            out_buf.at[slot], o_hbm.at[bi, pl.ds(si * CS, CS), :],
            out_sem.at[slot])

    for c in range(n_s):
        pltpu.make_async_copy(
            e_hbm.at[pl.ds(c * CS, CS), :],
            emb_buf.at[pl.ds(c * CS, CS), :], emb_sem.at[c]).start()
    for k in range(NBUF):
        in_copy(k, k).start()

    def step(t, carry):
        slot = lax.rem(t, NBUF)
        si = lax.rem(t, n_s)
        in_copy(t, slot).wait()

        @pl.when(t < n_s)
        def _():
            pltpu.make_async_copy(
                e_hbm.at[pl.ds(0, CS), :], emb_buf.at[pl.ds(0, CS), :],
                emb_sem.at[si]).wait()

        @pl.when(t >= NBUF)
        def _():
            out_copy(t - NBUF, slot).wait()

        out_buf[slot] = in_buf[slot] + emb_buf[pl.ds(si * CS, CS), :]
        out_copy(t, slot).start()

        @pl.when(t + NBUF < total)
        def _():
            in_copy(t + NBUF, slot).start()

        return carry

    lax.fori_loop(0, total, step, 0)

    for k in range(NBUF):
        slot = (total - NBUF + k) % NBUF
        out_copy(total - NBUF + k, slot).wait()


def _tc_add(inputs, emb):
    _, s, d = inputs.shape
    return pl.pallas_call(
        _tc_body,
        in_specs=[
            pl.BlockSpec(memory_space=pl.ANY),
            pl.BlockSpec(memory_space=pl.ANY),
        ],
        out_specs=pl.BlockSpec(memory_space=pl.ANY),
        out_shape=jax.ShapeDtypeStruct((TC_B, s, d), inputs.dtype),
        scratch_shapes=[
            pltpu.VMEM((NBUF, CS, d), jnp.float32),
            pltpu.VMEM((NBUF, CS, d), jnp.float32),
            pltpu.VMEM((s, d), jnp.float32),
            pltpu.SemaphoreType.DMA((NBUF,)),
            pltpu.SemaphoreType.DMA((NBUF,)),
            pltpu.SemaphoreType.DMA((s // CS,)),
        ],
    )(inputs, emb)


def kernel(inputs, embeddings):
    b, s, d = inputs.shape
    emb = embeddings[:s]
    sc_out = _sc_add_rows(inputs, emb, TC_B)
    tc_out = _tc_add(inputs, emb)
    return jnp.concatenate([tc_out, sc_out], axis=0)


# speedup vs baseline: 5.1820x; 2.2175x over previous
import jax
import jax.numpy as jnp
from jax.experimental import pallas as pl

BLOCK_S = 2048


def _add_body(x_ref, e_ref, o_ref):
    o_ref[0] = x_ref[0] + e_ref[...]


def kernel(inputs, embeddings):
    b, s, d = inputs.shape
    emb = embeddings[:s]
    grid = (s // BLOCK_S, b)
    return pl.pallas_call(
        _add_body,
        grid=grid,
        in_specs=[
            pl.BlockSpec((1, BLOCK_S, d), lambda i, j: (j, i, 0)),
            pl.BlockSpec((BLOCK_S, d), lambda i, j: (i, 0)),
        ],
        out_specs=pl.BlockSpec((1, BLOCK_S, d), lambda i, j: (j, i, 0)),
        out_shape=jax.ShapeDtypeStruct((b, s, d), inputs.dtype),
    )(inputs, emb)
